# baseline (speedup 1.0000x reference)
import functools

import jax
import jax.numpy as jnp
from jax.experimental import pallas as pl
from jax.experimental.pallas import tpu as pltpu


def _gat_body(atoms_ref, adjs_ref, e4_ref, w_ref, asrc_ref, adst_ref,
              wedge_ref, aedge_ref, bias_ref, out_ref, mv_ref, *, n, e):
    b = pl.program_id(0)
    nc = 128 // e
    nq = n * e // 128

    @pl.when(b == 0)
    def _build_mv():
        vec = jnp.sum(wedge_ref[...] * aedge_ref[...], axis=1, keepdims=True)
        r = jax.lax.broadcasted_iota(jnp.int32, (128, nc), 0)
        c = jax.lax.broadcasted_iota(jnp.int32, (128, nc), 1)
        grp = (r // e) == c
        m = jnp.zeros((128, nc), jnp.float32)
        for d in range(e):
            m = m + jnp.where(grp & ((r % e) == d), vec[d:d + 1, 0:1], 0.0)
        mv_ref[...] = m.astype(jnp.bfloat16)

    x = atoms_ref[0]
    xl = jnp.dot(x.astype(jnp.bfloat16), w_ref[...].astype(jnp.bfloat16),
                 preferred_element_type=jnp.float32)
    a_src = jax.lax.dot_general(xl, asrc_ref[...], (((1,), (1,)), ((), ())),
                                preferred_element_type=jnp.float32)
    a_dst = jax.lax.dot_general(adst_ref[...], xl, (((1,), (1,)), ((), ())),
                                preferred_element_type=jnp.float32)
    e4 = e4_ref[0].reshape(n, nq, 128)
    chunks = []
    for q in range(nq):
        eq = e4[:, q, :].astype(jnp.bfloat16)
        chunks.append(jnp.dot(eq, mv_ref[...],
                              preferred_element_type=jnp.float32))
    a_edge = jnp.concatenate(chunks, axis=1)

    logits = a_src + a_dst + a_edge
    logits = jnp.where(logits >= 0, logits, 0.2 * logits)
    mask = adjs_ref[0] > 0.5
    ml = jnp.where(mask, logits, -1e9)
    mx = jnp.max(ml, axis=0, keepdims=True)
    ex = jnp.exp(ml - mx)
    s = jnp.sum(ex, axis=0, keepdims=True)
    att = jnp.where(mask, ex / s, 0.0)
    out = jax.lax.dot_general(att.astype(jnp.bfloat16), xl.astype(jnp.bfloat16),
                              (((0,), (0,)), ((), ())),
                              preferred_element_type=jnp.float32)
    out_ref[0] = out + bias_ref[...]


def kernel(atoms, adjs, edges, W, att_src, att_dst, W_edge, att_edge, bias):
    B, N, D = atoms.shape
    E = edges.shape[-1]
    C = W.shape[-1]
    e4 = edges.reshape(B, N * E * N // 128, 128)
    w2 = W.reshape(D, C)
    wedge = W_edge.reshape(E, C)
    bias2 = bias.reshape(1, C)

    body = functools.partial(_gat_body, n=N, e=E)
    out = pl.pallas_call(
        body,
        grid=(B,),
        in_specs=[
            pl.BlockSpec((1, N, D), lambda b: (b, 0, 0)),
            pl.BlockSpec((1, N, N), lambda b: (b, 0, 0)),
            pl.BlockSpec((1, N * E * N // 128, 128), lambda b: (b, 0, 0)),
            pl.BlockSpec((D, C), lambda b: (0, 0)),
            pl.BlockSpec((1, C), lambda b: (0, 0)),
            pl.BlockSpec((1, C), lambda b: (0, 0)),
            pl.BlockSpec((E, C), lambda b: (0, 0)),
            pl.BlockSpec((1, C), lambda b: (0, 0)),
            pl.BlockSpec((1, C), lambda b: (0, 0)),
        ],
        out_specs=pl.BlockSpec((1, N, C), lambda b: (b, 0, 0)),
        out_shape=jax.ShapeDtypeStruct((B, N, C), jnp.float32),
        scratch_shapes=[pltpu.VMEM((128, 128 // E), jnp.bfloat16)],
    )(atoms, adjs, e4, w2, att_src, att_dst, wedge, att_edge, bias2)
    return out


# speedup vs baseline: 2.5809x; 2.5809x over previous
import functools

import jax
import jax.numpy as jnp
from jax.experimental import pallas as pl
from jax.experimental.pallas import tpu as pltpu


def _gat_body(atoms_ref, adjs_ref, ev_ref, w_ref, asrc_ref, adst_ref,
              wedge_ref, aedge_ref, bias_ref, out_ref, p_ref, *, n, e):
    b = pl.program_id(0)
    rows = n * 2 * e

    @pl.when(b == 0)
    def _build_p():
        vec = jnp.sum(wedge_ref[...] * aedge_ref[...], axis=1, keepdims=True)
        r = jax.lax.broadcasted_iota(jnp.int32, (rows, 2 * n), 0)
        c = jax.lax.broadcasted_iota(jnp.int32, (rows, 2 * n), 1)
        cond = ((r // (2 * e)) == (c % n)) & (((r // e) % 2) == (c // n))
        m = jnp.zeros((rows, 2 * n), jnp.float32)
        for d in range(e):
            m = m + jnp.where(cond & ((r % e) == d), vec[d:d + 1, 0:1], 0.0)
        p_ref[...] = m.astype(jnp.bfloat16)

    x = atoms_ref[0]
    xl = jnp.dot(x.astype(jnp.bfloat16), w_ref[...].astype(jnp.bfloat16),
                 preferred_element_type=jnp.float32)
    a_src = jax.lax.dot_general(xl, asrc_ref[...], (((1,), (1,)), ((), ())),
                                preferred_element_type=jnp.float32)
    a_dst = jax.lax.dot_general(adst_ref[...], xl, (((1,), (1,)), ((), ())),
                                preferred_element_type=jnp.float32)
    ev = ev_ref[...].astype(jnp.bfloat16)
    halves = [jax.lax.dot_general(p_ref[:, h * n:(h + 1) * n], ev,
                                  (((0,), (0,)), ((), ())),
                                  preferred_element_type=jnp.float32)
              for h in range(2)]
    a_edge = jnp.concatenate(halves, axis=1)

    logits = a_src + a_dst + a_edge
    logits = jnp.where(logits >= 0, logits, 0.2 * logits)
    mask = adjs_ref[0] > 0.5
    ml = jnp.where(mask, logits, -1e9)
    mx = jnp.max(ml, axis=0, keepdims=True)
    ex = jnp.exp(ml - mx)
    s = jnp.sum(ex, axis=0, keepdims=True)
    att = jnp.where(mask, ex / s, 0.0)
    out = jax.lax.dot_general(att.astype(jnp.bfloat16), xl.astype(jnp.bfloat16),
                              (((0,), (0,)), ((), ())),
                              preferred_element_type=jnp.float32)
    out_ref[0] = out + bias_ref[...]


def kernel(atoms, adjs, edges, W, att_src, att_dst, W_edge, att_edge, bias):
    B, N, D = atoms.shape
    E = edges.shape[-1]
    C = W.shape[-1]
    ev = edges.reshape(B, N, 2, 128, E).transpose(0, 1, 2, 4, 3)
    ev = ev.reshape(B * N * 2 * E, 128)
    rows = N * 2 * E
    w2 = W.reshape(D, C)
    wedge = W_edge.reshape(E, C)
    bias2 = bias.reshape(1, C)

    body = functools.partial(_gat_body, n=N, e=E)
    out = pl.pallas_call(
        body,
        grid=(B,),
        in_specs=[
            pl.BlockSpec((1, N, D), lambda b: (b, 0, 0)),
            pl.BlockSpec((1, N, N), lambda b: (b, 0, 0)),
            pl.BlockSpec((rows, 128), lambda b: (b, 0)),
            pl.BlockSpec((D, C), lambda b: (0, 0)),
            pl.BlockSpec((1, C), lambda b: (0, 0)),
            pl.BlockSpec((1, C), lambda b: (0, 0)),
            pl.BlockSpec((E, C), lambda b: (0, 0)),
            pl.BlockSpec((1, C), lambda b: (0, 0)),
            pl.BlockSpec((1, C), lambda b: (0, 0)),
        ],
        out_specs=pl.BlockSpec((1, N, C), lambda b: (b, 0, 0)),
        out_shape=jax.ShapeDtypeStruct((B, N, C), jnp.float32),
        scratch_shapes=[pltpu.VMEM((rows, 2 * N), jnp.bfloat16)],
    )(atoms, adjs, ev, w2, att_src, att_dst, wedge, att_edge, bias2)
    return out
